# baseline (speedup 1.0000x reference)
import functools
import jax
import jax.numpy as jnp
from jax import lax
from jax.experimental import pallas as pl
from jax.experimental.pallas import tpu as pltpu
from jax.experimental.pallas import tpu_sc as plsc

N, S, K = 4096, 200, 100
J = N * S
NW = 32
JW = J // NW
CB = 512
NSUB = JW // CB


def _scatter16(loc_ref, ids_v, sub, vals16):
    iota16 = lax.broadcasted_iota(jnp.int32, (16,), 0)
    zero16i = jnp.zeros((16,), jnp.int32)
    for g in range(CB // 16):
        ids16 = ids_v[pl.ds(sub * CB + g * 16, 16)]
        col16 = iota16 + (g * 16)
        plsc.store_scatter(loc_ref, [ids16, zero16i, col16], vals16)


def _sc_body(x_hbm, z_hbm, out_hbm, ids_v, loc, sem0, sem1):
    wid = lax.axis_index("s") * 2 + lax.axis_index("c")
    j0 = wid * JW

    pltpu.sync_copy(x_hbm.at[pl.ds(j0, JW)], ids_v)
    pltpu.sync_copy(z_hbm, loc.at[0])
    pltpu.sync_copy(z_hbm, loc.at[1])

    ones16 = jnp.ones((16,), jnp.float32)
    zeros16 = jnp.zeros((16,), jnp.float32)
    sems = (sem0, sem1)

    def step(t, carry):
        for phase in (0, 1):
            sub = 2 * t + phase

            @pl.when(t >= 1)
            def _recycle():
                pltpu.make_async_copy(
                    loc.at[phase],
                    out_hbm.at[:, pl.ds(wid, 1), pl.ds((sub - 2) * CB, CB)],
                    sems[phase],
                ).wait()
                _scatter16(loc.at[phase], ids_v, sub - 2, zeros16)

            _scatter16(loc.at[phase], ids_v, sub, ones16)
            pltpu.make_async_copy(
                loc.at[phase],
                out_hbm.at[:, pl.ds(wid, 1), pl.ds(sub * CB, CB)],
                sems[phase],
            ).start()
        return carry

    lax.fori_loop(0, NSUB // 2, step, 0)

    for phase in (0, 1):
        pltpu.make_async_copy(
            loc.at[phase],
            out_hbm.at[:, pl.ds(wid, 1), pl.ds((NSUB - 2 + phase) * CB, CB)],
            sems[phase],
        ).wait()


_sc_onehot = functools.partial(
    pl.kernel,
    out_type=jax.ShapeDtypeStruct((K, NW, JW), jnp.float32),
    mesh=plsc.VectorSubcoreMesh(core_axis_name="c", subcore_axis_name="s"),
    scratch_types=[
        pltpu.VMEM((JW,), jnp.int32),
        pltpu.VMEM((2, K, 1, CB), jnp.float32),
        pltpu.SemaphoreType.DMA,
        pltpu.SemaphoreType.DMA,
    ],
    compiler_params=pltpu.CompilerParams(needs_layout_passes=False),
)(_sc_body)


def kernel(inputs):
    x_flat = inputs.T.reshape(J)
    zeros = jnp.zeros((K, 1, CB), jnp.float32)
    out3 = _sc_onehot(x_flat, zeros)
    return jnp.transpose(out3.reshape(K, S, N), (2, 1, 0))


# speedup vs baseline: 3.5057x; 3.5057x over previous
import jax
import jax.numpy as jnp
from jax import lax
from jax.experimental import pallas as pl

N, S, K = 4096, 200, 100
K_BLK = 4
GRID = K // K_BLK


def _body(in_ref, out_ref):
    ids = in_ref[...]
    k0 = pl.program_id(0) * K_BLK
    for kk in range(K_BLK):
        out_ref[kk] = (ids == (k0 + kk)).astype(jnp.float32)


def kernel(inputs):
    x_t = inputs.T
    out_t = pl.pallas_call(
        _body,
        grid=(GRID,),
        in_specs=[pl.BlockSpec((S, N), lambda i: (0, 0))],
        out_specs=pl.BlockSpec((K_BLK, S, N), lambda i: (i, 0, 0)),
        out_shape=jax.ShapeDtypeStruct((K, S, N), jnp.float32),
    )(x_t)
    return jnp.transpose(out_t, (2, 1, 0))
